# baseline (speedup 1.0000x reference)
import dataclasses
import functools

import jax
import jax.numpy as jnp
from jax import lax
from jax.experimental import pallas as pl
from jax.experimental.pallas import tpu as pltpu
from jax.experimental.pallas import tpu_sc as plsc

_SC_CP = pltpu.CompilerParams()
if "needs_layout_passes" in pltpu.CompilerParams.__dataclass_fields__:
    _SC_CP = dataclasses.replace(_SC_CP, needs_layout_passes=False)

N = 10000
E = 320000
D = 128
H = 128
C = 40

NC = 2
NS = 16
NW = NC * NS
CHUNK = 128
CHUNKS_PER_TILE = 80
GROUP = 8
E_TILE = CHUNK * CHUNKS_PER_TILE
E_PAD = E_TILE * NW
N_CHUNKS = E_PAD // CHUNK
CC0 = 104
CC1 = 160 - CC0
N_PAD = 10240
ROWS_PER_TILE = N_PAD // NS


def _sc_deg_body(dst_hbm, out_hbm, dst_v, hist_v, sem):
    c = lax.axis_index("c")
    s = lax.axis_index("s")
    wid = c * NS + s
    pltpu.async_copy(dst_hbm.at[pl.ds(wid * E_TILE, E_TILE)], dst_v, sem).wait()

    zero16 = jnp.zeros((16,), jnp.float32)
    ones16 = jnp.ones((16,), jnp.float32)

    @pl.loop(0, N_PAD // 16)
    def _(i):
        hist_v[pl.ds(i * 16, 16)] = zero16

    @pl.loop(0, E_TILE // 16)
    def _(i):
        idx = dst_v[pl.ds(i * 16, 16)]
        plsc.addupdate_scatter(hist_v, [idx], ones16)

    pltpu.async_copy(hist_v, out_hbm.at[wid], sem).wait()


@jax.jit
def _sc_degree(dst_flat):
    kern = pl.kernel(
        _sc_deg_body,
        out_type=jax.ShapeDtypeStruct((NW, N_PAD), jnp.float32),
        mesh=plsc.VectorSubcoreMesh(core_axis_name="c", subcore_axis_name="s"),
        scratch_types=[
            pltpu.VMEM((E_TILE,), jnp.int32),
            pltpu.VMEM((N_PAD,), jnp.float32),
            pltpu.SemaphoreType.DMA,
        ],
        compiler_params=_SC_CP,
    )
    return kern(dst_flat)


def _edge_pass(y_hbm, src_hbm, dst_hbm, out_hbm,
               acc, src_v, dst_v, b0, b1, s0, s1, s, rbase, base, cc):
    zero16 = jnp.zeros((16,), jnp.float32)

    @pl.loop(0, CHUNK)
    def _(r):
        @pl.loop(0, H // 16)
        def _(l):
            b0[r, pl.ds(l * 16, 16)] = zero16

    @pl.loop(0, ROWS_PER_TILE // CHUNK)
    def _(i):
        pltpu.sync_copy(b0, acc.at[pl.ds(rbase + i * CHUNK, CHUNK)])

    plsc.subcore_barrier()

    @pl.loop(0, cc // GROUP)
    def _(g):
        gbase = base + g * GROUP
        pltpu.sync_copy(src_hbm.at[pl.ds(gbase, GROUP)], src_v)
        pltpu.sync_copy(dst_hbm.at[pl.ds(gbase, GROUP)], dst_v)

        @pl.loop(0, GROUP, step=2)
        def _(k):
            cpa = pltpu.async_copy(y_hbm.at[src_v.at[k]], b0, s0)
            cpb = pltpu.async_copy(y_hbm.at[src_v.at[k + 1]], b1, s1)
            cpa.wait()
            pltpu.sync_copy(b0, acc.at[dst_v.at[k]], add=True)
            cpb.wait()
            pltpu.sync_copy(b1, acc.at[dst_v.at[k + 1]], add=True)

    plsc.subcore_barrier()
    pltpu.sync_copy(acc.at[pl.ds(rbase, ROWS_PER_TILE)],
                    out_hbm.at[pl.ds(rbase, ROWS_PER_TILE)])


def _sc_scatter_body(y_hbm, src_hbm, dst_hbm, out_hbm,
                     acc, src_v, dst_v, b0, b1, s0, s1):
    c = lax.axis_index("c")
    s = lax.axis_index("s")
    rbase = s * ROWS_PER_TILE

    @pl.when(c == 0)
    def _():
        _edge_pass(y_hbm, src_hbm, dst_hbm, out_hbm.at[0],
                   acc, src_v, dst_v, b0, b1, s0, s1, s, rbase,
                   s * CC0, CC0)

    @pl.when(c == 1)
    def _():
        _edge_pass(y_hbm, src_hbm, dst_hbm, out_hbm.at[1],
                   acc, src_v, dst_v, b0, b1, s0, s1, s, rbase,
                   NS * CC0 + s * CC1, CC1)


def _make_sc_scatter():
    return pl.kernel(
        _sc_scatter_body,
        out_type=jax.ShapeDtypeStruct((NC, N_PAD, H), jnp.float32),
        mesh=plsc.VectorSubcoreMesh(core_axis_name="c", subcore_axis_name="s"),
        scratch_types=[
            pltpu.VMEM_SHARED((N_PAD, H), jnp.float32),
            pltpu.VMEM((GROUP, CHUNK), jnp.int32),
            pltpu.VMEM((GROUP, CHUNK), jnp.int32),
            pltpu.VMEM((CHUNK, H), jnp.float32),
            pltpu.VMEM((CHUNK, H), jnp.float32),
            pltpu.SemaphoreType.DMA,
            pltpu.SemaphoreType.DMA,
        ],
    )


_BLK = 1280
_GRID = N_PAD // _BLK

_DOT = functools.partial(
    lax.dot_general,
    dimension_numbers=(((1,), (0,)), ((), ())),
    precision=lax.Precision.HIGHEST,
    preferred_element_type=jnp.float32,
)


def _tc_first_body(part_ref, x_ref, w_ref, y_ref, dinv_ref):
    deg = jnp.sum(part_ref[...], axis=0) + 1.0
    dinv = lax.rsqrt(deg)
    y_ref[...] = _DOT(x_ref[...], w_ref[...]) * dinv[:, None]
    dinv_ref[...] = dinv[:, None]


def _tc_first(partials, x, W0):
    return pl.pallas_call(
        _tc_first_body,
        grid=(_GRID,),
        in_specs=[
            pl.BlockSpec((NW, _BLK), lambda j: (0, j)),
            pl.BlockSpec((_BLK, D), lambda j: (j, 0)),
            pl.BlockSpec((D, H), lambda j: (0, 0)),
        ],
        out_specs=[
            pl.BlockSpec((_BLK, H), lambda j: (j, 0)),
            pl.BlockSpec((_BLK, 1), lambda j: (j, 0)),
        ],
        out_shape=[
            jax.ShapeDtypeStruct((N_PAD, H), jnp.float32),
            jax.ShapeDtypeStruct((N_PAD, 1), jnp.float32),
        ],
    )(partials, x, W0)


_AGG_IN_SPECS = [
    pl.BlockSpec((NC, _BLK, H), lambda j: (0, j, 0)),
    pl.BlockSpec((_BLK, H), lambda j: (j, 0)),
    pl.BlockSpec((_BLK, 1), lambda j: (j, 0)),
    pl.BlockSpec((1, H), lambda j: (0, 0)),
    pl.BlockSpec((H, H), lambda j: (0, 0)),
]


def _tc_mid_body(agg_ref, y_ref, dinv_ref, b_ref, w_ref, ynext_ref):
    a = agg_ref[0] + agg_ref[1] + y_ref[...]
    h = jnp.maximum(a * dinv_ref[...] + b_ref[...], 0.0)
    ynext_ref[...] = _DOT(h, w_ref[...]) * dinv_ref[...]


def _tc_mid(agg, y, dinv, b, Wn):
    return pl.pallas_call(
        _tc_mid_body,
        grid=(_GRID,),
        in_specs=_AGG_IN_SPECS,
        out_specs=pl.BlockSpec((_BLK, H), lambda j: (j, 0)),
        out_shape=jax.ShapeDtypeStruct((N_PAD, H), jnp.float32),
    )(agg, y, dinv, b, Wn)


def _tc_final_body(agg_ref, y_ref, dinv_ref, b_ref, wl_ref, bl_ref, out_ref):
    a = agg_ref[0] + agg_ref[1] + y_ref[...]
    h = jnp.maximum(a * dinv_ref[...] + b_ref[...], 0.0)
    out_ref[...] = _DOT(h, wl_ref[...]) + bl_ref[...]


def _tc_final(agg, y, dinv, b, Wl_pad, bl_pad):
    return pl.pallas_call(
        _tc_final_body,
        grid=(_GRID,),
        in_specs=_AGG_IN_SPECS + [pl.BlockSpec((1, H), lambda j: (0, 0))],
        out_specs=pl.BlockSpec((_BLK, H), lambda j: (j, 0)),
        out_shape=jax.ShapeDtypeStruct((N_PAD, H), jnp.float32),
    )(agg, y, dinv, b, Wl_pad, bl_pad)


def kernel(x, edge_index, W0, b0, W1, b1, W2, b2, Wl, bl):
    src = edge_index[0]
    dst = edge_index[1]
    pad = E_PAD - E
    src_p = jnp.concatenate([src, jnp.zeros((pad,), jnp.int32)])
    dst_p = jnp.concatenate([dst, jnp.full((pad,), N, jnp.int32)])
    src3 = src_p.reshape(N_CHUNKS, CHUNK)
    dst3 = dst_p.reshape(N_CHUNKS, CHUNK)
    x_pad = jnp.pad(x, ((0, N_PAD - N), (0, 0)))

    partials = _sc_degree(dst_p)

    y0, dinv = _tc_first(partials, x_pad, W0)

    sc_scatter = _make_sc_scatter()
    agg0 = sc_scatter(y0, src3, dst3)
    y1 = _tc_mid(agg0, y0, dinv, b0.reshape(1, H), W1)
    agg1 = sc_scatter(y1, src3, dst3)
    y2 = _tc_mid(agg1, y1, dinv, b1.reshape(1, H), W2)
    agg2 = sc_scatter(y2, src3, dst3)

    Wl_pad = jnp.pad(Wl, ((0, 0), (0, H - C)))
    bl_pad = jnp.pad(bl, ((0, H - C))).reshape(1, H)
    out = _tc_final(agg2, y2, dinv, b2.reshape(1, H), Wl_pad, bl_pad)
    return out[:N, :C]


# speedup vs baseline: 1.1471x; 1.1471x over previous
import dataclasses
import functools

import jax
import jax.numpy as jnp
from jax import lax
from jax.experimental import pallas as pl
from jax.experimental.pallas import tpu as pltpu
from jax.experimental.pallas import tpu_sc as plsc

_SC_CP = pltpu.CompilerParams()
if "needs_layout_passes" in pltpu.CompilerParams.__dataclass_fields__:
    _SC_CP = dataclasses.replace(_SC_CP, needs_layout_passes=False)

N = 10000
E = 320000
D = 128
H = 128
C = 40

NC = 2
NS = 16
NW = NC * NS
CHUNK = 128
CHUNKS_PER_TILE = 80
GROUP = 8
E_TILE = CHUNK * CHUNKS_PER_TILE
E_PAD = E_TILE * NW
N_CHUNKS = E_PAD // CHUNK
CC0 = 128
CC1 = 160 - CC0
N_PAD = 10240
ROWS_PER_TILE = N_PAD // NS


def _sc_deg_body(dst_hbm, out_hbm, dst_v, hist_v, sem):
    c = lax.axis_index("c")
    s = lax.axis_index("s")
    wid = c * NS + s
    pltpu.async_copy(dst_hbm.at[pl.ds(wid * E_TILE, E_TILE)], dst_v, sem).wait()

    zero16 = jnp.zeros((16,), jnp.float32)
    ones16 = jnp.ones((16,), jnp.float32)

    @pl.loop(0, N_PAD // 16)
    def _(i):
        hist_v[pl.ds(i * 16, 16)] = zero16

    @pl.loop(0, E_TILE // 16)
    def _(i):
        idx = dst_v[pl.ds(i * 16, 16)]
        plsc.addupdate_scatter(hist_v, [idx], ones16)

    pltpu.async_copy(hist_v, out_hbm.at[wid], sem).wait()


@jax.jit
def _sc_degree(dst_flat):
    kern = pl.kernel(
        _sc_deg_body,
        out_type=jax.ShapeDtypeStruct((NW, N_PAD), jnp.float32),
        mesh=plsc.VectorSubcoreMesh(core_axis_name="c", subcore_axis_name="s"),
        scratch_types=[
            pltpu.VMEM((E_TILE,), jnp.int32),
            pltpu.VMEM((N_PAD,), jnp.float32),
            pltpu.SemaphoreType.DMA,
        ],
        compiler_params=_SC_CP,
    )
    return kern(dst_flat)


def _edge_pass(y_hbm, src_hbm, dst_hbm, zeros_hbm, out_hbm,
               acc, src_v, dst_v, b0, b1, s0, s1, s, rbase, base, cc):
    pltpu.async_copy(zeros_hbm.at[pl.ds(rbase, ROWS_PER_TILE)],
                     acc.at[pl.ds(rbase, ROWS_PER_TILE)], s0).wait()
    plsc.subcore_barrier()

    @pl.loop(0, cc // GROUP)
    def _(g):
        gbase = base + g * GROUP
        pltpu.sync_copy(src_hbm.at[pl.ds(gbase, GROUP)], src_v)
        pltpu.sync_copy(dst_hbm.at[pl.ds(gbase, GROUP)], dst_v)

        @pl.loop(0, GROUP, step=2)
        def _(k):
            cpa = pltpu.async_copy(y_hbm.at[src_v.at[k]], b0, s0)
            cpb = pltpu.async_copy(y_hbm.at[src_v.at[k + 1]], b1, s1)
            cpa.wait()
            pltpu.sync_copy(b0, acc.at[dst_v.at[k]], add=True)
            cpb.wait()
            pltpu.sync_copy(b1, acc.at[dst_v.at[k + 1]], add=True)

    plsc.subcore_barrier()
    pltpu.sync_copy(acc.at[pl.ds(rbase, ROWS_PER_TILE)],
                    out_hbm.at[pl.ds(rbase, ROWS_PER_TILE)])


def _sc_scatter_body(y_hbm, src_hbm, dst_hbm, zeros_hbm, out_hbm,
                     acc, src_v, dst_v, b0, b1, s0, s1):
    c = lax.axis_index("c")
    s = lax.axis_index("s")
    rbase = s * ROWS_PER_TILE

    @pl.when(c == 0)
    def _():
        _edge_pass(y_hbm, src_hbm, dst_hbm, zeros_hbm, out_hbm.at[0],
                   acc, src_v, dst_v, b0, b1, s0, s1, s, rbase,
                   s * CC0, CC0)

    @pl.when(c == 1)
    def _():
        _edge_pass(y_hbm, src_hbm, dst_hbm, zeros_hbm, out_hbm.at[1],
                   acc, src_v, dst_v, b0, b1, s0, s1, s, rbase,
                   NS * CC0 + s * CC1, CC1)


def _make_sc_scatter():
    return pl.kernel(
        _sc_scatter_body,
        out_type=jax.ShapeDtypeStruct((NC, N_PAD, H), jnp.float32),
        mesh=plsc.VectorSubcoreMesh(core_axis_name="c", subcore_axis_name="s"),
        scratch_types=[
            pltpu.VMEM_SHARED((N_PAD, H), jnp.float32),
            pltpu.VMEM((GROUP, CHUNK), jnp.int32),
            pltpu.VMEM((GROUP, CHUNK), jnp.int32),
            pltpu.VMEM((CHUNK, H), jnp.float32),
            pltpu.VMEM((CHUNK, H), jnp.float32),
            pltpu.SemaphoreType.DMA,
            pltpu.SemaphoreType.DMA,
        ],
    )


_BLK = 1280
_GRID = N_PAD // _BLK

_DOT = functools.partial(
    lax.dot_general,
    dimension_numbers=(((1,), (0,)), ((), ())),
    precision=lax.Precision.HIGHEST,
    preferred_element_type=jnp.float32,
)


def _tc_first_body(part_ref, x_ref, w_ref, y_ref, dinv_ref):
    deg = jnp.sum(part_ref[...], axis=0) + 1.0
    dinv = lax.rsqrt(deg)
    y_ref[...] = _DOT(x_ref[...], w_ref[...]) * dinv[:, None]
    dinv_ref[...] = dinv[:, None]


def _tc_first(partials, x, W0):
    return pl.pallas_call(
        _tc_first_body,
        grid=(_GRID,),
        in_specs=[
            pl.BlockSpec((NW, _BLK), lambda j: (0, j)),
            pl.BlockSpec((_BLK, D), lambda j: (j, 0)),
            pl.BlockSpec((D, H), lambda j: (0, 0)),
        ],
        out_specs=[
            pl.BlockSpec((_BLK, H), lambda j: (j, 0)),
            pl.BlockSpec((_BLK, 1), lambda j: (j, 0)),
        ],
        out_shape=[
            jax.ShapeDtypeStruct((N_PAD, H), jnp.float32),
            jax.ShapeDtypeStruct((N_PAD, 1), jnp.float32),
        ],
    )(partials, x, W0)


_AGG_IN_SPECS = [
    pl.BlockSpec((NC, _BLK, H), lambda j: (0, j, 0)),
    pl.BlockSpec((_BLK, H), lambda j: (j, 0)),
    pl.BlockSpec((_BLK, 1), lambda j: (j, 0)),
    pl.BlockSpec((1, H), lambda j: (0, 0)),
    pl.BlockSpec((H, H), lambda j: (0, 0)),
]


def _tc_mid_body(agg_ref, y_ref, dinv_ref, b_ref, w_ref, ynext_ref):
    a = agg_ref[0] + agg_ref[1] + y_ref[...]
    h = jnp.maximum(a * dinv_ref[...] + b_ref[...], 0.0)
    ynext_ref[...] = _DOT(h, w_ref[...]) * dinv_ref[...]


def _tc_mid(agg, y, dinv, b, Wn):
    return pl.pallas_call(
        _tc_mid_body,
        grid=(_GRID,),
        in_specs=_AGG_IN_SPECS,
        out_specs=pl.BlockSpec((_BLK, H), lambda j: (j, 0)),
        out_shape=jax.ShapeDtypeStruct((N_PAD, H), jnp.float32),
    )(agg, y, dinv, b, Wn)


def _tc_final_body(agg_ref, y_ref, dinv_ref, b_ref, wl_ref, bl_ref, out_ref):
    a = agg_ref[0] + agg_ref[1] + y_ref[...]
    h = jnp.maximum(a * dinv_ref[...] + b_ref[...], 0.0)
    out_ref[...] = _DOT(h, wl_ref[...]) + bl_ref[...]


def _tc_final(agg, y, dinv, b, Wl_pad, bl_pad):
    return pl.pallas_call(
        _tc_final_body,
        grid=(_GRID,),
        in_specs=_AGG_IN_SPECS + [pl.BlockSpec((1, H), lambda j: (0, 0))],
        out_specs=pl.BlockSpec((_BLK, H), lambda j: (j, 0)),
        out_shape=jax.ShapeDtypeStruct((N_PAD, H), jnp.float32),
    )(agg, y, dinv, b, Wl_pad, bl_pad)


def kernel(x, edge_index, W0, b0, W1, b1, W2, b2, Wl, bl):
    src = edge_index[0]
    dst = edge_index[1]
    pad = E_PAD - E
    src_p = jnp.concatenate([src, jnp.zeros((pad,), jnp.int32)])
    dst_p = jnp.concatenate([dst, jnp.full((pad,), N, jnp.int32)])
    src3 = src_p.reshape(N_CHUNKS, CHUNK)
    dst3 = dst_p.reshape(N_CHUNKS, CHUNK)
    zeros_hbm = jnp.zeros((N_PAD, H), jnp.float32)
    x_pad = jnp.pad(x, ((0, N_PAD - N), (0, 0)))

    partials = _sc_degree(dst_p)

    y0, dinv = _tc_first(partials, x_pad, W0)

    sc_scatter = _make_sc_scatter()
    agg0 = sc_scatter(y0, src3, dst3, zeros_hbm)
    y1 = _tc_mid(agg0, y0, dinv, b0.reshape(1, H), W1)
    agg1 = sc_scatter(y1, src3, dst3, zeros_hbm)
    y2 = _tc_mid(agg1, y1, dinv, b1.reshape(1, H), W2)
    agg2 = sc_scatter(y2, src3, dst3, zeros_hbm)

    Wl_pad = jnp.pad(Wl, ((0, 0), (0, H - C)))
    bl_pad = jnp.pad(bl, ((0, H - C))).reshape(1, H)
    out = _tc_final(agg2, y2, dinv, b2.reshape(1, H), Wl_pad, bl_pad)
    return out[:N, :C]


# speedup vs baseline: 1.1755x; 1.0248x over previous
import dataclasses
import functools

import jax
import jax.numpy as jnp
from jax import lax
from jax.experimental import pallas as pl
from jax.experimental.pallas import tpu as pltpu
from jax.experimental.pallas import tpu_sc as plsc

_SC_CP = pltpu.CompilerParams()
if "needs_layout_passes" in pltpu.CompilerParams.__dataclass_fields__:
    _SC_CP = dataclasses.replace(_SC_CP, needs_layout_passes=False)

N = 10000
E = 320000
D = 128
H = 128
C = 40

NC = 2
NS = 16
NW = NC * NS
CHUNK = 128
CHUNKS_PER_TILE = 80
GROUP = 8
E_TILE = CHUNK * CHUNKS_PER_TILE
E_PAD = E_TILE * NW
N_CHUNKS = E_PAD // CHUNK
CC0 = 144
CC1 = 160 - CC0
N_PAD = 10240
ROWS_PER_TILE = N_PAD // NS


def _sc_deg_body(dst_hbm, out_hbm, dst_v, hist_v, sem):
    c = lax.axis_index("c")
    s = lax.axis_index("s")
    wid = c * NS + s
    pltpu.async_copy(dst_hbm.at[pl.ds(wid * E_TILE, E_TILE)], dst_v, sem).wait()

    zero16 = jnp.zeros((16,), jnp.float32)
    ones16 = jnp.ones((16,), jnp.float32)

    @pl.loop(0, N_PAD // 16)
    def _(i):
        hist_v[pl.ds(i * 16, 16)] = zero16

    @pl.loop(0, E_TILE // 16)
    def _(i):
        idx = dst_v[pl.ds(i * 16, 16)]
        plsc.addupdate_scatter(hist_v, [idx], ones16)

    pltpu.async_copy(hist_v, out_hbm.at[wid], sem).wait()


@jax.jit
def _sc_degree(dst_flat):
    kern = pl.kernel(
        _sc_deg_body,
        out_type=jax.ShapeDtypeStruct((NW, N_PAD), jnp.float32),
        mesh=plsc.VectorSubcoreMesh(core_axis_name="c", subcore_axis_name="s"),
        scratch_types=[
            pltpu.VMEM((E_TILE,), jnp.int32),
            pltpu.VMEM((N_PAD,), jnp.float32),
            pltpu.SemaphoreType.DMA,
        ],
        compiler_params=_SC_CP,
    )
    return kern(dst_flat)


def _edge_pass(y_hbm, src_hbm, dst_hbm, zeros_hbm, out_hbm,
               acc, src_v, dst_v, b0, b1, s0, s1, s, rbase, base, cc):
    pltpu.async_copy(zeros_hbm.at[pl.ds(rbase, ROWS_PER_TILE)],
                     acc.at[pl.ds(rbase, ROWS_PER_TILE)], s0).wait()
    plsc.subcore_barrier()

    @pl.loop(0, cc // GROUP)
    def _(g):
        gbase = base + g * GROUP
        pltpu.sync_copy(src_hbm.at[pl.ds(gbase, GROUP)], src_v)
        pltpu.sync_copy(dst_hbm.at[pl.ds(gbase, GROUP)], dst_v)

        @pl.loop(0, GROUP, step=2)
        def _(k):
            cpa = pltpu.async_copy(y_hbm.at[src_v.at[k]], b0, s0)
            cpb = pltpu.async_copy(y_hbm.at[src_v.at[k + 1]], b1, s1)
            cpa.wait()
            pltpu.sync_copy(b0, acc.at[dst_v.at[k]], add=True)
            cpb.wait()
            pltpu.sync_copy(b1, acc.at[dst_v.at[k + 1]], add=True)

    plsc.subcore_barrier()
    pltpu.sync_copy(acc.at[pl.ds(rbase, ROWS_PER_TILE)],
                    out_hbm.at[pl.ds(rbase, ROWS_PER_TILE)])


def _sc_scatter_body(y_hbm, src_hbm, dst_hbm, zeros_hbm, out_hbm,
                     acc, src_v, dst_v, b0, b1, s0, s1):
    c = lax.axis_index("c")
    s = lax.axis_index("s")
    rbase = s * ROWS_PER_TILE

    @pl.when(c == 0)
    def _():
        _edge_pass(y_hbm, src_hbm, dst_hbm, zeros_hbm, out_hbm.at[0],
                   acc, src_v, dst_v, b0, b1, s0, s1, s, rbase,
                   s * CC0, CC0)

    @pl.when(c == 1)
    def _():
        _edge_pass(y_hbm, src_hbm, dst_hbm, zeros_hbm, out_hbm.at[1],
                   acc, src_v, dst_v, b0, b1, s0, s1, s, rbase,
                   NS * CC0 + s * CC1, CC1)


def _make_sc_scatter():
    return pl.kernel(
        _sc_scatter_body,
        out_type=jax.ShapeDtypeStruct((NC, N_PAD, H), jnp.float32),
        mesh=plsc.VectorSubcoreMesh(core_axis_name="c", subcore_axis_name="s"),
        scratch_types=[
            pltpu.VMEM_SHARED((N_PAD, H), jnp.float32),
            pltpu.VMEM((GROUP, CHUNK), jnp.int32),
            pltpu.VMEM((GROUP, CHUNK), jnp.int32),
            pltpu.VMEM((CHUNK, H), jnp.float32),
            pltpu.VMEM((CHUNK, H), jnp.float32),
            pltpu.SemaphoreType.DMA,
            pltpu.SemaphoreType.DMA,
        ],
    )


_BLK = 1280
_GRID = N_PAD // _BLK

_DOT = functools.partial(
    lax.dot_general,
    dimension_numbers=(((1,), (0,)), ((), ())),
    precision=lax.Precision.HIGHEST,
    preferred_element_type=jnp.float32,
)


def _tc_first_body(part_ref, x_ref, w_ref, y_ref, dinv_ref):
    deg = jnp.sum(part_ref[...], axis=0) + 1.0
    dinv = lax.rsqrt(deg)
    y_ref[...] = _DOT(x_ref[...], w_ref[...]) * dinv[:, None]
    dinv_ref[...] = dinv[:, None]


def _tc_first(partials, x, W0):
    return pl.pallas_call(
        _tc_first_body,
        grid=(_GRID,),
        in_specs=[
            pl.BlockSpec((NW, _BLK), lambda j: (0, j)),
            pl.BlockSpec((_BLK, D), lambda j: (j, 0)),
            pl.BlockSpec((D, H), lambda j: (0, 0)),
        ],
        out_specs=[
            pl.BlockSpec((_BLK, H), lambda j: (j, 0)),
            pl.BlockSpec((_BLK, 1), lambda j: (j, 0)),
        ],
        out_shape=[
            jax.ShapeDtypeStruct((N_PAD, H), jnp.float32),
            jax.ShapeDtypeStruct((N_PAD, 1), jnp.float32),
        ],
    )(partials, x, W0)


_AGG_IN_SPECS = [
    pl.BlockSpec((NC, _BLK, H), lambda j: (0, j, 0)),
    pl.BlockSpec((_BLK, H), lambda j: (j, 0)),
    pl.BlockSpec((_BLK, 1), lambda j: (j, 0)),
    pl.BlockSpec((1, H), lambda j: (0, 0)),
    pl.BlockSpec((H, H), lambda j: (0, 0)),
]


def _tc_mid_body(agg_ref, y_ref, dinv_ref, b_ref, w_ref, ynext_ref):
    a = agg_ref[0] + agg_ref[1] + y_ref[...]
    h = jnp.maximum(a * dinv_ref[...] + b_ref[...], 0.0)
    ynext_ref[...] = _DOT(h, w_ref[...]) * dinv_ref[...]


def _tc_mid(agg, y, dinv, b, Wn):
    return pl.pallas_call(
        _tc_mid_body,
        grid=(_GRID,),
        in_specs=_AGG_IN_SPECS,
        out_specs=pl.BlockSpec((_BLK, H), lambda j: (j, 0)),
        out_shape=jax.ShapeDtypeStruct((N_PAD, H), jnp.float32),
    )(agg, y, dinv, b, Wn)


def _tc_final_body(agg_ref, y_ref, dinv_ref, b_ref, wl_ref, bl_ref, out_ref):
    a = agg_ref[0] + agg_ref[1] + y_ref[...]
    h = jnp.maximum(a * dinv_ref[...] + b_ref[...], 0.0)
    out_ref[...] = _DOT(h, wl_ref[...]) + bl_ref[...]


def _tc_final(agg, y, dinv, b, Wl_pad, bl_pad):
    return pl.pallas_call(
        _tc_final_body,
        grid=(_GRID,),
        in_specs=_AGG_IN_SPECS + [pl.BlockSpec((1, H), lambda j: (0, 0))],
        out_specs=pl.BlockSpec((_BLK, H), lambda j: (j, 0)),
        out_shape=jax.ShapeDtypeStruct((N_PAD, H), jnp.float32),
    )(agg, y, dinv, b, Wl_pad, bl_pad)


def kernel(x, edge_index, W0, b0, W1, b1, W2, b2, Wl, bl):
    src = edge_index[0]
    dst = edge_index[1]
    pad = E_PAD - E
    src_p = jnp.concatenate([src, jnp.zeros((pad,), jnp.int32)])
    dst_p = jnp.concatenate([dst, jnp.full((pad,), N, jnp.int32)])
    src3 = src_p.reshape(N_CHUNKS, CHUNK)
    dst3 = dst_p.reshape(N_CHUNKS, CHUNK)
    zeros_hbm = jnp.zeros((N_PAD, H), jnp.float32)
    x_pad = jnp.pad(x, ((0, N_PAD - N), (0, 0)))

    partials = _sc_degree(dst_p)

    y0, dinv = _tc_first(partials, x_pad, W0)

    sc_scatter = _make_sc_scatter()
    agg0 = sc_scatter(y0, src3, dst3, zeros_hbm)
    y1 = _tc_mid(agg0, y0, dinv, b0.reshape(1, H), W1)
    agg1 = sc_scatter(y1, src3, dst3, zeros_hbm)
    y2 = _tc_mid(agg1, y1, dinv, b1.reshape(1, H), W2)
    agg2 = sc_scatter(y2, src3, dst3, zeros_hbm)

    Wl_pad = jnp.pad(Wl, ((0, 0), (0, H - C)))
    bl_pad = jnp.pad(bl, ((0, H - C))).reshape(1, H)
    out = _tc_final(agg2, y2, dinv, b2.reshape(1, H), Wl_pad, bl_pad)
    return out[:N, :C]


# speedup vs baseline: 1.1842x; 1.0074x over previous
import dataclasses
import functools

import jax
import jax.numpy as jnp
from jax import lax
from jax.experimental import pallas as pl
from jax.experimental.pallas import tpu as pltpu
from jax.experimental.pallas import tpu_sc as plsc

_SC_CP = pltpu.CompilerParams()
if "needs_layout_passes" in pltpu.CompilerParams.__dataclass_fields__:
    _SC_CP = dataclasses.replace(_SC_CP, needs_layout_passes=False)

N = 10000
E = 320000
D = 128
H = 128
C = 40

NC = 2
NS = 16
NW = NC * NS
CHUNK = 128
CHUNKS_PER_TILE = 80
GROUP = 8
E_TILE = CHUNK * CHUNKS_PER_TILE
E_PAD = E_TILE * NW
N_CHUNKS = E_PAD // CHUNK
CC0 = 152
CC1 = 160 - CC0
N_PAD = 10240
ROWS_PER_TILE = N_PAD // NS


def _sc_deg_body(dst_hbm, out_hbm, dst_v, hist_v, sem):
    c = lax.axis_index("c")
    s = lax.axis_index("s")
    wid = c * NS + s
    pltpu.async_copy(dst_hbm.at[pl.ds(wid * E_TILE, E_TILE)], dst_v, sem).wait()

    zero16 = jnp.zeros((16,), jnp.float32)
    ones16 = jnp.ones((16,), jnp.float32)

    @pl.loop(0, N_PAD // 16)
    def _(i):
        hist_v[pl.ds(i * 16, 16)] = zero16

    @pl.loop(0, E_TILE // 16)
    def _(i):
        idx = dst_v[pl.ds(i * 16, 16)]
        plsc.addupdate_scatter(hist_v, [idx], ones16)

    pltpu.async_copy(hist_v, out_hbm.at[wid], sem).wait()


@jax.jit
def _sc_degree(dst_flat):
    kern = pl.kernel(
        _sc_deg_body,
        out_type=jax.ShapeDtypeStruct((NW, N_PAD), jnp.float32),
        mesh=plsc.VectorSubcoreMesh(core_axis_name="c", subcore_axis_name="s"),
        scratch_types=[
            pltpu.VMEM((E_TILE,), jnp.int32),
            pltpu.VMEM((N_PAD,), jnp.float32),
            pltpu.SemaphoreType.DMA,
        ],
        compiler_params=_SC_CP,
    )
    return kern(dst_flat)


def _edge_pass(y_hbm, src_hbm, dst_hbm, zeros_hbm, out_hbm,
               acc, src_v, dst_v, b0, b1, s0, s1, s, rbase, base, cc,
               zero_local):
    if zero_local:
        zero16 = jnp.zeros((16,), jnp.float32)

        @pl.loop(0, CHUNK)
        def _(r):
            @pl.loop(0, H // 16)
            def _(l):
                b0[r, pl.ds(l * 16, 16)] = zero16

        @pl.loop(0, ROWS_PER_TILE // CHUNK)
        def _(i):
            pltpu.sync_copy(b0, acc.at[pl.ds(rbase + i * CHUNK, CHUNK)])
    else:
        pltpu.async_copy(zeros_hbm.at[pl.ds(rbase, ROWS_PER_TILE)],
                         acc.at[pl.ds(rbase, ROWS_PER_TILE)], s0).wait()
    plsc.subcore_barrier()

    @pl.loop(0, cc // GROUP)
    def _(g):
        gbase = base + g * GROUP
        pltpu.sync_copy(src_hbm.at[pl.ds(gbase, GROUP)], src_v)
        pltpu.sync_copy(dst_hbm.at[pl.ds(gbase, GROUP)], dst_v)

        @pl.loop(0, GROUP, step=2)
        def _(k):
            cpa = pltpu.async_copy(y_hbm.at[src_v.at[k]], b0, s0)
            cpb = pltpu.async_copy(y_hbm.at[src_v.at[k + 1]], b1, s1)
            cpa.wait()
            pltpu.sync_copy(b0, acc.at[dst_v.at[k]], add=True)
            cpb.wait()
            pltpu.sync_copy(b1, acc.at[dst_v.at[k + 1]], add=True)

    plsc.subcore_barrier()
    cps = []
    for i in range(ROWS_PER_TILE // CHUNK):
        r0 = rbase + i * CHUNK
        cps.append(pltpu.async_copy(acc.at[pl.ds(r0, CHUNK)],
                                    out_hbm.at[pl.ds(r0, CHUNK)],
                                    s1 if i % 2 else s0))
    for cp in cps:
        cp.wait()


def _sc_scatter_body(y_hbm, src_hbm, dst_hbm, zeros_hbm, out_hbm,
                     acc, src_v, dst_v, b0, b1, s0, s1):
    c = lax.axis_index("c")
    s = lax.axis_index("s")
    rbase = s * ROWS_PER_TILE

    @pl.when(c == 0)
    def _():
        _edge_pass(y_hbm, src_hbm, dst_hbm, zeros_hbm, out_hbm.at[0],
                   acc, src_v, dst_v, b0, b1, s0, s1, s, rbase,
                   s * CC0, CC0, False)

    @pl.when(c == 1)
    def _():
        _edge_pass(y_hbm, src_hbm, dst_hbm, zeros_hbm, out_hbm.at[1],
                   acc, src_v, dst_v, b0, b1, s0, s1, s, rbase,
                   NS * CC0 + s * CC1, CC1, True)


def _make_sc_scatter():
    return pl.kernel(
        _sc_scatter_body,
        out_type=jax.ShapeDtypeStruct((NC, N_PAD, H), jnp.float32),
        mesh=plsc.VectorSubcoreMesh(core_axis_name="c", subcore_axis_name="s"),
        scratch_types=[
            pltpu.VMEM_SHARED((N_PAD, H), jnp.float32),
            pltpu.VMEM((GROUP, CHUNK), jnp.int32),
            pltpu.VMEM((GROUP, CHUNK), jnp.int32),
            pltpu.VMEM((CHUNK, H), jnp.float32),
            pltpu.VMEM((CHUNK, H), jnp.float32),
            pltpu.SemaphoreType.DMA,
            pltpu.SemaphoreType.DMA,
        ],
    )


_BLK = 1280
_GRID = N_PAD // _BLK

_DOT = functools.partial(
    lax.dot_general,
    dimension_numbers=(((1,), (0,)), ((), ())),
    precision=lax.Precision.HIGHEST,
    preferred_element_type=jnp.float32,
)


def _tc_first_body(part_ref, x_ref, w_ref, y_ref, dinv_ref):
    deg = jnp.sum(part_ref[...], axis=0) + 1.0
    dinv = lax.rsqrt(deg)
    y_ref[...] = _DOT(x_ref[...], w_ref[...]) * dinv[:, None]
    dinv_ref[...] = dinv[:, None]


def _tc_first(partials, x, W0):
    return pl.pallas_call(
        _tc_first_body,
        grid=(_GRID,),
        in_specs=[
            pl.BlockSpec((NW, _BLK), lambda j: (0, j)),
            pl.BlockSpec((_BLK, D), lambda j: (j, 0)),
            pl.BlockSpec((D, H), lambda j: (0, 0)),
        ],
        out_specs=[
            pl.BlockSpec((_BLK, H), lambda j: (j, 0)),
            pl.BlockSpec((_BLK, 1), lambda j: (j, 0)),
        ],
        out_shape=[
            jax.ShapeDtypeStruct((N_PAD, H), jnp.float32),
            jax.ShapeDtypeStruct((N_PAD, 1), jnp.float32),
        ],
    )(partials, x, W0)


_AGG_IN_SPECS = [
    pl.BlockSpec((NC, _BLK, H), lambda j: (0, j, 0)),
    pl.BlockSpec((_BLK, H), lambda j: (j, 0)),
    pl.BlockSpec((_BLK, 1), lambda j: (j, 0)),
    pl.BlockSpec((1, H), lambda j: (0, 0)),
    pl.BlockSpec((H, H), lambda j: (0, 0)),
]


def _tc_mid_body(agg_ref, y_ref, dinv_ref, b_ref, w_ref, ynext_ref):
    a = agg_ref[0] + agg_ref[1] + y_ref[...]
    h = jnp.maximum(a * dinv_ref[...] + b_ref[...], 0.0)
    ynext_ref[...] = _DOT(h, w_ref[...]) * dinv_ref[...]


def _tc_mid(agg, y, dinv, b, Wn):
    return pl.pallas_call(
        _tc_mid_body,
        grid=(_GRID,),
        in_specs=_AGG_IN_SPECS,
        out_specs=pl.BlockSpec((_BLK, H), lambda j: (j, 0)),
        out_shape=jax.ShapeDtypeStruct((N_PAD, H), jnp.float32),
    )(agg, y, dinv, b, Wn)


def _tc_final_body(agg_ref, y_ref, dinv_ref, b_ref, wl_ref, bl_ref, out_ref):
    a = agg_ref[0] + agg_ref[1] + y_ref[...]
    h = jnp.maximum(a * dinv_ref[...] + b_ref[...], 0.0)
    out_ref[...] = _DOT(h, wl_ref[...]) + bl_ref[...]


def _tc_final(agg, y, dinv, b, Wl_pad, bl_pad):
    return pl.pallas_call(
        _tc_final_body,
        grid=(_GRID,),
        in_specs=_AGG_IN_SPECS + [pl.BlockSpec((1, H), lambda j: (0, 0))],
        out_specs=pl.BlockSpec((_BLK, H), lambda j: (j, 0)),
        out_shape=jax.ShapeDtypeStruct((N_PAD, H), jnp.float32),
    )(agg, y, dinv, b, Wl_pad, bl_pad)


def kernel(x, edge_index, W0, b0, W1, b1, W2, b2, Wl, bl):
    src = edge_index[0]
    dst = edge_index[1]
    pad = E_PAD - E
    src_p = jnp.concatenate([src, jnp.zeros((pad,), jnp.int32)])
    dst_p = jnp.concatenate([dst, jnp.full((pad,), N, jnp.int32)])
    src3 = src_p.reshape(N_CHUNKS, CHUNK)
    dst3 = dst_p.reshape(N_CHUNKS, CHUNK)
    zeros_hbm = jnp.zeros((N_PAD, H), jnp.float32)
    x_pad = jnp.pad(x, ((0, N_PAD - N), (0, 0)))

    partials = _sc_degree(dst_p)

    y0, dinv = _tc_first(partials, x_pad, W0)

    sc_scatter = _make_sc_scatter()
    agg0 = sc_scatter(y0, src3, dst3, zeros_hbm)
    y1 = _tc_mid(agg0, y0, dinv, b0.reshape(1, H), W1)
    agg1 = sc_scatter(y1, src3, dst3, zeros_hbm)
    y2 = _tc_mid(agg1, y1, dinv, b1.reshape(1, H), W2)
    agg2 = sc_scatter(y2, src3, dst3, zeros_hbm)

    Wl_pad = jnp.pad(Wl, ((0, 0), (0, H - C)))
    bl_pad = jnp.pad(bl, ((0, H - C))).reshape(1, H)
    out = _tc_final(agg2, y2, dinv, b2.reshape(1, H), Wl_pad, bl_pad)
    return out[:N, :C]
